# Initial kernel scaffold; baseline (speedup 1.0000x reference)
import jax
import jax.numpy as jnp
from jax.experimental import pallas as pl


def kernel(x0, x1, edge_index, enc0_W1, enc0_b1, enc0_g1, enc0_bb1, enc0_W2, enc0_b2, enc0_rg, enc0_rb, enc1_W1, enc1_b1, enc1_g1, enc1_bb1, enc1_W2, enc1_b2, enc1_rg, enc1_rb, comb_W, comb_b, comb_g, comb_bb, dec0_W, dec0_b, dec1_W, dec1_b, clf_W1, clf_b1, clf_W2, clf_b2):
    raise NotImplementedError("write your pallas kernel here")


# speedup vs baseline: 20.4511x; 20.4511x over previous
import functools

import jax
import jax.numpy as jnp
from jax import lax
from jax.experimental import pallas as pl
from jax.experimental.pallas import tpu as pltpu
from jax.experimental.pallas import tpu_sc as plsc

NN = 100000
EE = 1600000
NW = 32
BATCH = 128
STEPS = 400
OUTER = 25
G = 16
EPAD = STEPS * BATCH * NW
NPAD = 100352
ZCH = NPAD // (16 * BATCH)
DOUT = 100096
BN = 2000


def _gelu(x):
    return 0.5 * x * (1.0 + lax.erf(x * 0.7071067811865476))


def _ln(x, g, b, eps=1e-5):
    m = jnp.mean(x, axis=-1, keepdims=True)
    v = jnp.mean((x - m) ** 2, axis=-1, keepdims=True)
    return (x - m) / jnp.sqrt(v + eps) * g + b


@functools.lru_cache(maxsize=None)
def _get_sc_degree():
    mesh = plsc.VectorSubcoreMesh(core_axis_name="c", subcore_axis_name="s")
    return pl.kernel(
        _sc_degree_body,
        out_type=jax.ShapeDtypeStruct((2 * DOUT,), jnp.float32),
        mesh=mesh,
        compiler_params=pltpu.CompilerParams(use_tc_tiling_on_sc=False),
        scratch_types=[
            pltpu.VMEM_SHARED((NPAD,), jnp.float32),
            pltpu.VMEM((G, BATCH), jnp.int32),
            pltpu.VMEM((BATCH,), jnp.float32),
            pltpu.VMEM((BATCH,), jnp.float32),
            pltpu.VMEM((368,), jnp.float32),
        ],
    )


def _sc_degree_body(dst_hbm, out_hbm, acc, didx, ones_v, zero_v, bounce):
    c = lax.axis_index("c")
    s = lax.axis_index("s")
    wid = c * 16 + s
    for r in range(BATCH // 16):
        ones_v[pl.ds(r * 16, 16)] = jnp.ones((16,), jnp.float32)
        zero_v[pl.ds(r * 16, 16)] = jnp.zeros((16,), jnp.float32)

    def zb(i, carry):
        pltpu.sync_copy(zero_v, acc.at[pl.ds((s * ZCH + i) * BATCH, BATCH)])
        return carry

    lax.fori_loop(0, ZCH, zb, 0)
    plsc.subcore_barrier()

    def outer(g, carry):
        pltpu.sync_copy(dst_hbm.at[pl.ds(wid * STEPS + g * G, G)], didx)

        def body(j, carry2):
            pltpu.sync_copy(ones_v, acc.at[didx.at[j]], add=True)
            return carry2

        return lax.fori_loop(0, G, body, carry)

    lax.fori_loop(0, OUTER, outer, 0)
    plsc.subcore_barrier()
    def ocp(i, carry):
        pltpu.sync_copy(acc.at[pl.ds(s * 6256 + i * 368, 368)], bounce)
        pltpu.sync_copy(bounce, out_hbm.at[pl.ds(c * DOUT + s * 6256 + i * 368, 368)])
        return carry

    lax.fori_loop(0, 17, ocp, 0)


@functools.lru_cache(maxsize=None)
def _get_sc_round():
    mesh = plsc.VectorSubcoreMesh(core_axis_name="c", subcore_axis_name="s")
    return pl.kernel(
        _sc_round_body,
        out_type=jax.ShapeDtypeStruct((2, DOUT, 16), jnp.float32),
        mesh=mesh,
        compiler_params=pltpu.CompilerParams(use_tc_tiling_on_sc=False),
        scratch_types=[
            pltpu.VMEM_SHARED((NPAD, 16), jnp.float32),
            pltpu.VMEM((G, BATCH), jnp.int32),
            pltpu.VMEM((G, BATCH), jnp.int32),
            pltpu.VMEM((BATCH, 16), jnp.float32),
            pltpu.VMEM((BATCH, 16), jnp.float32),
            pltpu.VMEM((368, 16), jnp.float32),
            pltpu.SemaphoreType.DMA,
        ],
    )


def _sc_round_body(tab_hbm, src_hbm, dst_hbm, out_hbm, acc, sidx, didx, zbuf, rows, bounce, sem):
    c = lax.axis_index("c")
    s = lax.axis_index("s")
    wid = c * 16 + s
    for r in range(BATCH):
        zbuf[r] = jnp.zeros((16,), jnp.float32)

    def zb(i, carry):
        pltpu.sync_copy(zbuf, acc.at[pl.ds((s * ZCH + i) * BATCH, BATCH)])
        return carry

    lax.fori_loop(0, ZCH, zb, 0)
    plsc.subcore_barrier()

    def outer(g, carry):
        pltpu.sync_copy(src_hbm.at[pl.ds(wid * STEPS + g * G, G)], sidx)
        pltpu.sync_copy(dst_hbm.at[pl.ds(wid * STEPS + g * G, G)], didx)

        def body(j, carry2):
            pltpu.async_copy(tab_hbm.at[sidx.at[j]], rows, sem).wait()
            pltpu.sync_copy(rows, acc.at[didx.at[j]], add=True)
            return carry2

        return lax.fori_loop(0, G, body, carry)

    lax.fori_loop(0, OUTER, outer, 0)
    plsc.subcore_barrier()
    def ocp(i, carry):
        pltpu.sync_copy(acc.at[pl.ds(s * 6256 + i * 368, 368)], bounce)
        pltpu.sync_copy(bounce, out_hbm.at[c, pl.ds(s * 6256 + i * 368, 368)])
        return carry

    lax.fori_loop(0, 17, ocp, 0)


def _enc_body(x0_r, x1_r, dga_r, dgb_r,
              w01, b01, g01, bb01, w02, b02, rg0, rb0,
              w11, b11, g11, bb11, w12, b12, rg1, rb1,
              cw0, cw1, cb, t1_r):
    def enc(x, W1, b1, g1, bb1, W2, b2, rg, rb):
        h = jnp.dot(x, W1[...], preferred_element_type=jnp.float32) + b1[...]
        h = _ln(h, g1[...], bb1[...])
        h = _gelu(h)
        h2 = jnp.dot(h, W2[...], preferred_element_type=jnp.float32) + b2[...]
        y = h2 + h2
        return _ln(y, rg[...], rb[...])

    z0 = enc(x0_r[...], w01, b01, g01, bb01, w02, b02, rg0, rb0)
    z1 = enc(x1_r[...], w11, b11, g11, bb11, w12, b12, rg1, rb1)
    u = (jnp.dot(z0, cw0[...], preferred_element_type=jnp.float32)
         + jnp.dot(z1, cw1[...], preferred_element_type=jnp.float32) + cb[...])
    dinv = lax.rsqrt(dga_r[...] + dgb_r[...] + 1.0)
    t1_r[...] = u * dinv


def _mid_body(s1_r, t1_r, dga_r, dgb_r, cg, cbb, cb, w1, b1, w2, b2, z_r, t2_r, dom_r):
    dinv = lax.rsqrt(dga_r[...] + dgb_r[...] + 1.0)
    s1 = s1_r[...]
    zp = (s1[0] + s1[1] + t1_r[...]) * dinv + cb[...]
    z = _ln(zp, cg[...], cbb[...])
    z_r[...] = z
    t2_r[...] = z * dinv
    h = _gelu(jnp.dot(z, w1[...], preferred_element_type=jnp.float32) + b1[...])
    dom_r[...] = jnp.dot(h, w2[...], preferred_element_type=jnp.float32) + b2[...]


def _dec_body(s2_r, t2_r, dga_r, dgb_r, d0w, d0b, d1w, d1b, rec0_r, rec1_r):
    dinv = lax.rsqrt(dga_r[...] + dgb_r[...] + 1.0)
    s2 = s2_r[...]
    agg = (s2[0] + s2[1] + t2_r[...]) * dinv
    rec0_r[...] = jnp.dot(agg, d0w[...], preferred_element_type=jnp.float32) + d0b[...]
    rec1_r[...] = jnp.dot(agg, d1w[...], preferred_element_type=jnp.float32) + d1b[...]


def _full(shape):
    return pl.BlockSpec(shape, lambda i: tuple(0 for _ in shape))


def _rows(width):
    return pl.BlockSpec((BN, width), lambda i: (i, 0))


_GRID = NN // BN
_PART_SPEC = pl.BlockSpec((2, BN, 16), lambda i: (0, i, 0))


def kernel(x0, x1, edge_index, enc0_W1, enc0_b1, enc0_g1, enc0_bb1, enc0_W2,
           enc0_b2, enc0_rg, enc0_rb, enc1_W1, enc1_b1, enc1_g1, enc1_bb1,
           enc1_W2, enc1_b2, enc1_rg, enc1_rb, comb_W, comb_b, comb_g,
           comb_bb, dec0_W, dec0_b, dec1_W, dec1_b, clf_W1, clf_b1, clf_W2,
           clf_b2):
    src = jnp.concatenate(
        [edge_index[0], jnp.zeros((EPAD - EE,), jnp.int32)]).reshape(
            NW * STEPS, BATCH)
    dst = jnp.concatenate(
        [edge_index[1], jnp.full((EPAD - EE,), NN, jnp.int32)]).reshape(
            NW * STEPS, BATCH)

    degp = _get_sc_degree()(dst)
    dga = degp[:NN, None]
    dgb = degp[DOUT:DOUT + NN, None]

    row = lambda a: a[None, :]
    t1 = pl.pallas_call(
        _enc_body,
        grid=(_GRID,),
        in_specs=[
            _rows(128), _rows(128), _rows(1), _rows(1),
            _full((128, 64)), _full((1, 64)), _full((1, 64)), _full((1, 64)),
            _full((64, 16)), _full((1, 16)), _full((1, 16)), _full((1, 16)),
            _full((128, 64)), _full((1, 64)), _full((1, 64)), _full((1, 64)),
            _full((64, 16)), _full((1, 16)), _full((1, 16)), _full((1, 16)),
            _full((16, 16)), _full((16, 16)), _full((1, 16)),
        ],
        out_specs=_rows(16),
        out_shape=jax.ShapeDtypeStruct((NN, 16), jnp.float32),
    )(x0, x1, dga, dgb,
      enc0_W1, row(enc0_b1), row(enc0_g1), row(enc0_bb1),
      enc0_W2, row(enc0_b2), row(enc0_rg), row(enc0_rb),
      enc1_W1, row(enc1_b1), row(enc1_g1), row(enc1_bb1),
      enc1_W2, row(enc1_b2), row(enc1_rg), row(enc1_rb),
      comb_W[:16], comb_W[16:], row(comb_b))

    s1 = _get_sc_round()(t1, src, dst)[:, :NN]

    z, t2, dom = pl.pallas_call(
        _mid_body,
        grid=(_GRID,),
        in_specs=[
            _PART_SPEC, _rows(16), _rows(1), _rows(1),
            _full((1, 16)), _full((1, 16)), _full((1, 16)),
            _full((16, 64)), _full((1, 64)),
            _full((64, 8)), _full((1, 8)),
        ],
        out_specs=[_rows(16), _rows(16), _rows(8)],
        out_shape=[
            jax.ShapeDtypeStruct((NN, 16), jnp.float32),
            jax.ShapeDtypeStruct((NN, 16), jnp.float32),
            jax.ShapeDtypeStruct((NN, 8), jnp.float32),
        ],
    )(s1, t1, dga, dgb, row(comb_g), row(comb_bb), row(comb_b),
      clf_W1, row(clf_b1), clf_W2, row(clf_b2))

    s2 = _get_sc_round()(t2, src, dst)[:, :NN]

    rec0, rec1 = pl.pallas_call(
        _dec_body,
        grid=(_GRID,),
        in_specs=[
            _PART_SPEC, _rows(16), _rows(1), _rows(1),
            _full((16, 128)), _full((1, 128)),
            _full((16, 128)), _full((1, 128)),
        ],
        out_specs=[_rows(128), _rows(128)],
        out_shape=[
            jax.ShapeDtypeStruct((NN, 128), jnp.float32),
            jax.ShapeDtypeStruct((NN, 128), jnp.float32),
        ],
    )(s2, t2, dga, dgb, dec0_W, row(dec0_b), dec1_W, row(dec1_b))

    return z, rec0, rec1, dom


# speedup vs baseline: 28.7023x; 1.4035x over previous
import functools

import jax
import jax.numpy as jnp
from jax import lax
from jax.experimental import pallas as pl
from jax.experimental.pallas import tpu as pltpu
from jax.experimental.pallas import tpu_sc as plsc

NN = 100000
EE = 1600000
NW = 32
BATCH = 128
STEPS = 400
OUTER = 25
G = 16
EPAD = STEPS * BATCH * NW
NPAD = 100352
ZCH = NPAD // (16 * BATCH)
DOUT = 100096
BN = 2048


def _gelu(x):
    return 0.5 * x * (1.0 + lax.erf(x * 0.7071067811865476))


def _ln(x, g, b, eps=1e-5):
    m = jnp.mean(x, axis=-1, keepdims=True)
    v = jnp.mean((x - m) ** 2, axis=-1, keepdims=True)
    return (x - m) / jnp.sqrt(v + eps) * g + b


@functools.lru_cache(maxsize=None)
def _get_sc_degree():
    mesh = plsc.VectorSubcoreMesh(core_axis_name="c", subcore_axis_name="s")
    return pl.kernel(
        _sc_degree_body,
        out_type=[jax.ShapeDtypeStruct((DOUT,), jnp.float32),
                  jax.ShapeDtypeStruct((DOUT,), jnp.float32)],
        mesh=mesh,
        compiler_params=pltpu.CompilerParams(use_tc_tiling_on_sc=False),
        scratch_types=[
            pltpu.VMEM_SHARED((NPAD,), jnp.float32),
            pltpu.VMEM((G, BATCH), jnp.int32),
            pltpu.VMEM((BATCH,), jnp.float32),
            pltpu.VMEM((BATCH,), jnp.float32),
            pltpu.VMEM((368,), jnp.float32),
        ],
    )


def _sc_degree_body(dst_hbm, out0_hbm, out1_hbm, acc, didx, ones_v, zero_v, bounce):
    c = lax.axis_index("c")
    s = lax.axis_index("s")
    wid = c * 16 + s
    for r in range(BATCH // 16):
        ones_v[pl.ds(r * 16, 16)] = jnp.ones((16,), jnp.float32)
        zero_v[pl.ds(r * 16, 16)] = jnp.zeros((16,), jnp.float32)

    def zb(i, carry):
        pltpu.sync_copy(zero_v, acc.at[pl.ds((s * ZCH + i) * BATCH, BATCH)])
        return carry

    lax.fori_loop(0, ZCH, zb, 0)
    plsc.subcore_barrier()

    def outer(g, carry):
        pltpu.sync_copy(dst_hbm.at[pl.ds(wid * STEPS + g * G, G)], didx)

        def body(j, carry2):
            pltpu.sync_copy(ones_v, acc.at[didx.at[j]], add=True)
            return carry2

        return lax.fori_loop(0, G, body, carry)

    lax.fori_loop(0, OUTER, outer, 0)
    plsc.subcore_barrier()
    def ocp(i, carry):
        pltpu.sync_copy(acc.at[pl.ds(s * 6256 + i * 368, 368)], bounce)

        @pl.when(c == 0)
        def _():
            pltpu.sync_copy(bounce, out0_hbm.at[pl.ds(s * 6256 + i * 368, 368)])

        @pl.when(c == 1)
        def _():
            pltpu.sync_copy(bounce, out1_hbm.at[pl.ds(s * 6256 + i * 368, 368)])

        return carry

    lax.fori_loop(0, 17, ocp, 0)


@functools.lru_cache(maxsize=None)
def _get_sc_round():
    mesh = plsc.VectorSubcoreMesh(core_axis_name="c", subcore_axis_name="s")
    return pl.kernel(
        _sc_round_body,
        out_type=jax.ShapeDtypeStruct((2, DOUT, 16), jnp.float32),
        mesh=mesh,
        compiler_params=pltpu.CompilerParams(use_tc_tiling_on_sc=False),
        scratch_types=[
            pltpu.VMEM_SHARED((NPAD, 16), jnp.float32),
            pltpu.VMEM((G, BATCH), jnp.int32),
            pltpu.VMEM((G, BATCH), jnp.int32),
            pltpu.VMEM((BATCH, 16), jnp.float32),
            pltpu.VMEM((BATCH, 16), jnp.float32),
            pltpu.VMEM((BATCH, 16), jnp.float32),
            pltpu.VMEM((368, 16), jnp.float32),
            pltpu.SemaphoreType.DMA,
            pltpu.SemaphoreType.DMA,
        ],
    )


def _sc_round_body(tab_hbm, src_hbm, dst_hbm, out_hbm, acc, sidx, didx, zbuf,
                   rows_a, rows_b, bounce, sem_a, sem_b):
    c = lax.axis_index("c")
    s = lax.axis_index("s")
    wid = c * 16 + s
    for r in range(BATCH):
        zbuf[r] = jnp.zeros((16,), jnp.float32)

    def zb(i, carry):
        pltpu.sync_copy(zbuf, acc.at[pl.ds((s * ZCH + i) * BATCH, BATCH)])
        return carry

    lax.fori_loop(0, ZCH, zb, 0)
    plsc.subcore_barrier()

    bufs = (rows_a, rows_b)
    sems = (sem_a, sem_b)

    def outer(g, carry):
        pltpu.sync_copy(src_hbm.at[pl.ds(wid * STEPS + g * G, G)], sidx)
        pltpu.sync_copy(dst_hbm.at[pl.ds(wid * STEPS + g * G, G)], didx)
        pend = pltpu.async_copy(tab_hbm.at[sidx.at[0]], bufs[0], sems[0])
        for j in range(G):
            if j + 1 < G:
                nxt = pltpu.async_copy(
                    tab_hbm.at[sidx.at[j + 1]], bufs[(j + 1) % 2], sems[(j + 1) % 2])
            pend.wait()
            pltpu.sync_copy(bufs[j % 2], acc.at[didx.at[j]], add=True)
            if j + 1 < G:
                pend = nxt
        return carry

    lax.fori_loop(0, OUTER, outer, 0)
    plsc.subcore_barrier()
    def ocp(i, carry):
        pltpu.sync_copy(acc.at[pl.ds(s * 6256 + i * 368, 368)], bounce)
        pltpu.sync_copy(bounce, out_hbm.at[c, pl.ds(s * 6256 + i * 368, 368)])
        return carry

    lax.fori_loop(0, 17, ocp, 0)


def _enc_body(x0_r, x1_r, dga_r, dgb_r,
              w01, b01, g01, bb01, w02, b02, rg0, rb0,
              w11, b11, g11, bb11, w12, b12, rg1, rb1,
              cw0, cw1, cb, t1_r):
    def enc(x, W1, b1, g1, bb1, W2, b2, rg, rb):
        h = jnp.dot(x, W1[...], preferred_element_type=jnp.float32) + b1[...]
        h = _ln(h, g1[...], bb1[...])
        h = _gelu(h)
        h2 = jnp.dot(h, W2[...], preferred_element_type=jnp.float32) + b2[...]
        y = h2 + h2
        return _ln(y, rg[...], rb[...])

    z0 = enc(x0_r[...], w01, b01, g01, bb01, w02, b02, rg0, rb0)
    z1 = enc(x1_r[...], w11, b11, g11, bb11, w12, b12, rg1, rb1)
    u = (jnp.dot(z0, cw0[...], preferred_element_type=jnp.float32)
         + jnp.dot(z1, cw1[...], preferred_element_type=jnp.float32) + cb[...])
    dinv = lax.rsqrt(dga_r[...] + dgb_r[...] + 1.0)[:, None]
    t1_r[...] = u * dinv


def _mid_body(s1_r, t1_r, dga_r, dgb_r, cg, cbb, cb, w1, b1, w2, b2, z_r, t2_r, dom_r):
    dinv = lax.rsqrt(dga_r[...] + dgb_r[...] + 1.0)[:, None]
    s1 = s1_r[...]
    zp = (s1[0] + s1[1] + t1_r[...]) * dinv + cb[...]
    z = _ln(zp, cg[...], cbb[...])
    z_r[...] = z
    t2_r[...] = z * dinv
    h = _gelu(jnp.dot(z, w1[...], preferred_element_type=jnp.float32) + b1[...])
    dom_r[...] = jnp.dot(h, w2[...], preferred_element_type=jnp.float32) + b2[...]


def _dec_body(s2_r, t2_r, dga_r, dgb_r, d0w, d0b, d1w, d1b, rec0_r, rec1_r):
    dinv = lax.rsqrt(dga_r[...] + dgb_r[...] + 1.0)[:, None]
    s2 = s2_r[...]
    agg = (s2[0] + s2[1] + t2_r[...]) * dinv
    rec0_r[...] = jnp.dot(agg, d0w[...], preferred_element_type=jnp.float32) + d0b[...]
    rec1_r[...] = jnp.dot(agg, d1w[...], preferred_element_type=jnp.float32) + d1b[...]


def _full(shape):
    return pl.BlockSpec(shape, lambda i: tuple(0 for _ in shape))


def _rows(width):
    return pl.BlockSpec((BN, width), lambda i: (i, 0))


_GRID = -(-NN // BN)
_PART_SPEC = pl.BlockSpec((2, BN, 16), lambda i: (0, i, 0))
_DEG1 = pl.BlockSpec((BN,), lambda i: (i,))


def kernel(x0, x1, edge_index, enc0_W1, enc0_b1, enc0_g1, enc0_bb1, enc0_W2,
           enc0_b2, enc0_rg, enc0_rb, enc1_W1, enc1_b1, enc1_g1, enc1_bb1,
           enc1_W2, enc1_b2, enc1_rg, enc1_rb, comb_W, comb_b, comb_g,
           comb_bb, dec0_W, dec0_b, dec1_W, dec1_b, clf_W1, clf_b1, clf_W2,
           clf_b2):
    src = jnp.concatenate(
        [edge_index[0], jnp.zeros((EPAD - EE,), jnp.int32)]).reshape(
            NW * STEPS, BATCH)
    dst = jnp.concatenate(
        [edge_index[1], jnp.full((EPAD - EE,), NN, jnp.int32)]).reshape(
            NW * STEPS, BATCH)

    dga, dgb = _get_sc_degree()(dst)

    row = lambda a: a[None, :]
    t1 = pl.pallas_call(
        _enc_body,
        grid=(_GRID,),
        in_specs=[
            _rows(128), _rows(128), _DEG1, _DEG1,
            _full((128, 64)), _full((1, 64)), _full((1, 64)), _full((1, 64)),
            _full((64, 16)), _full((1, 16)), _full((1, 16)), _full((1, 16)),
            _full((128, 64)), _full((1, 64)), _full((1, 64)), _full((1, 64)),
            _full((64, 16)), _full((1, 16)), _full((1, 16)), _full((1, 16)),
            _full((16, 16)), _full((16, 16)), _full((1, 16)),
        ],
        out_specs=_rows(16),
        out_shape=jax.ShapeDtypeStruct((NN, 16), jnp.float32),
    )(x0, x1, dga, dgb,
      enc0_W1, row(enc0_b1), row(enc0_g1), row(enc0_bb1),
      enc0_W2, row(enc0_b2), row(enc0_rg), row(enc0_rb),
      enc1_W1, row(enc1_b1), row(enc1_g1), row(enc1_bb1),
      enc1_W2, row(enc1_b2), row(enc1_rg), row(enc1_rb),
      comb_W[:16], comb_W[16:], row(comb_b))

    s1 = _get_sc_round()(t1, src, dst)

    z, t2, dom = pl.pallas_call(
        _mid_body,
        grid=(_GRID,),
        in_specs=[
            _PART_SPEC, _rows(16), _DEG1, _DEG1,
            _full((1, 16)), _full((1, 16)), _full((1, 16)),
            _full((16, 64)), _full((1, 64)),
            _full((64, 8)), _full((1, 8)),
        ],
        out_specs=[_rows(16), _rows(16), _rows(8)],
        out_shape=[
            jax.ShapeDtypeStruct((NN, 16), jnp.float32),
            jax.ShapeDtypeStruct((NN, 16), jnp.float32),
            jax.ShapeDtypeStruct((NN, 8), jnp.float32),
        ],
    )(s1, t1, dga, dgb, row(comb_g), row(comb_bb), row(comb_b),
      clf_W1, row(clf_b1), clf_W2, row(clf_b2))

    s2 = _get_sc_round()(t2, src, dst)

    rec0, rec1 = pl.pallas_call(
        _dec_body,
        grid=(_GRID,),
        in_specs=[
            _PART_SPEC, _rows(16), _DEG1, _DEG1,
            _full((16, 128)), _full((1, 128)),
            _full((16, 128)), _full((1, 128)),
        ],
        out_specs=[_rows(128), _rows(128)],
        out_shape=[
            jax.ShapeDtypeStruct((NN, 128), jnp.float32),
            jax.ShapeDtypeStruct((NN, 128), jnp.float32),
        ],
    )(s2, t2, dga, dgb, dec0_W, row(dec0_b), dec1_W, row(dec1_b))

    return z, rec0, rec1, dom
